# baseline (speedup 1.0000x reference)
import functools
import jax
import jax.numpy as jnp
from jax import lax
from jax.experimental import pallas as pl
from jax.experimental.pallas import tpu as pltpu, tpu_sc as plsc

B, S, D = 4, 8192, 768
S_TC = 7168
S_SC = S - S_TC
BS = 1024

L = 16
W = 128
RW = D // W
NW = 32
PPW = S_SC // 8
RPW = PPW * RW
CR = 128
G = RPW // CR
K = G // 3

_mesh = plsc.VectorSubcoreMesh(core_axis_name="c", subcore_axis_name="s")


def _add_kernel(x_ref, t_ref, o_ref):
    o_ref[...] = x_ref[...] + t_ref[...]


@functools.partial(
    pl.kernel,
    mesh=_mesh,
    out_type=jax.ShapeDtypeStruct((B * S_SC * RW, W), jnp.float32),
    scratch_types=[
        pltpu.VMEM((CR, W), jnp.float32), pltpu.VMEM((CR, W), jnp.float32),
        pltpu.VMEM((CR, W), jnp.float32),
        pltpu.VMEM((CR, W), jnp.float32), pltpu.VMEM((CR, W), jnp.float32),
        pltpu.VMEM((CR, W), jnp.float32),
        pltpu.SemaphoreType.DMA, pltpu.SemaphoreType.DMA,
        pltpu.SemaphoreType.DMA,
        pltpu.SemaphoreType.DMA, pltpu.SemaphoreType.DMA,
        pltpu.SemaphoreType.DMA,
    ],
)
def _sc_add(x_hbm, t_hbm, out_hbm,
            xb0, xb1, xb2, tb0, tb1, tb2,
            si0, si1, si2, so0, so1, so2):
    w = lax.axis_index("c") * 16 + lax.axis_index("s")
    b = lax.div(w, 8)
    r8 = lax.rem(w, 8)
    s0 = S_TC + r8 * PPW
    x_base = (b * S + s0) * RW
    t_base = s0 * RW
    o_base = (b * S_SC + r8 * PPW) * RW
    xbs = (xb0, xb1, xb2)
    tbs = (tb0, tb1, tb2)
    sis = (si0, si1, si2)
    sos = (so0, so1, so2)

    def start_in(c, p):
        o = c * CR
        pltpu.async_copy(x_hbm.at[pl.ds(x_base + o, CR), :], xbs[p], sis[p])
        pltpu.async_copy(t_hbm.at[pl.ds(t_base + o, CR), :], tbs[p], sis[p])

    def wait_in(p):
        pltpu.make_async_copy(x_hbm.at[pl.ds(0, CR), :], xbs[p], sis[p]).wait()
        pltpu.make_async_copy(t_hbm.at[pl.ds(0, CR), :], tbs[p], sis[p]).wait()

    def start_out(c, p):
        pltpu.async_copy(xbs[p], out_hbm.at[pl.ds(o_base + c * CR, CR), :],
                         sos[p])

    def wait_out(p):
        pltpu.make_async_copy(xbs[p], out_hbm.at[pl.ds(0, CR), :],
                              sos[p]).wait()

    def compute(p):
        xb, tb = xbs[p], tbs[p]

        def rows2(i, _):
            r = i * 2
            for rr in (0, 1):
                for j in range(W // L):
                    sl = pl.ds(j * L, L)
                    xb[r + rr, sl] = xb[r + rr, sl] + tb[r + rr, sl]
            return 0

        lax.fori_loop(0, CR // 2, rows2, 0)

    start_in(0, 0)
    start_in(1, 1)

    def macro(k, _):
        c = k * 3

        @pl.when(k > 0)
        def _():
            wait_out(2)

        start_in(c + 2, 2)
        wait_in(0)
        compute(0)
        start_out(c, 0)

        @pl.when(k < K - 1)
        def _():
            wait_out(0)
            start_in(c + 3, 0)

        wait_in(1)
        compute(1)
        start_out(c + 1, 1)

        @pl.when(k < K - 1)
        def _():
            wait_out(1)
            start_in(c + 4, 1)

        wait_in(2)
        compute(2)
        start_out(c + 2, 2)
        return 0

    lax.fori_loop(0, K, macro, 0)
    wait_out(0)
    wait_out(1)
    wait_out(2)


def kernel(x, embed_table):
    tc_out = pl.pallas_call(
        _add_kernel,
        grid=(S_TC // BS, B),
        in_specs=[
            pl.BlockSpec((1, BS, D), lambda s, b: (b, s, 0)),
            pl.BlockSpec((BS, D), lambda s, b: (s, 0)),
        ],
        out_specs=pl.BlockSpec((1, BS, D), lambda s, b: (b, s, 0)),
        out_shape=jax.ShapeDtypeStruct((B, S_TC, D), x.dtype),
    )(x, embed_table)

    xf = x.reshape(B * S * RW, W)
    tf = embed_table.reshape(-1, W)
    sc_out = _sc_add(xf, tf).reshape(B, S_SC, D)
    return jnp.concatenate([tc_out, sc_out], axis=1)


# speedup vs baseline: 4.1413x; 4.1413x over previous
import jax
import jax.numpy as jnp
from jax.experimental import pallas as pl

_BS = 2048


def _add_kernel(x_ref, t_ref, o_ref):
    o_ref[...] = x_ref[...] + t_ref[...]


def kernel(x, embed_table):
    B, S, D = x.shape
    grid = (S // _BS, B)
    return pl.pallas_call(
        _add_kernel,
        grid=grid,
        in_specs=[
            pl.BlockSpec((1, _BS, D), lambda s, b: (b, s, 0)),
            pl.BlockSpec((_BS, D), lambda s, b: (s, 0)),
        ],
        out_specs=pl.BlockSpec((1, _BS, D), lambda s, b: (b, s, 0)),
        out_shape=jax.ShapeDtypeStruct((B, S, D), x.dtype),
    )(x, embed_table)
